# baseline (speedup 1.0000x reference)
import functools

import jax
import jax.numpy as jnp
from jax import lax
from jax.experimental import pallas as pl
from jax.experimental.pallas import tpu as pltpu
from jax.experimental.pallas import tpu_sc as plsc

V = 100000
D = 64
B = 1024
CTX = 20

NC = 2
NS = 16
NW = NC * NS
DHALF = D // 2
NLANE = 16
NGRP = B // NLANE

_SC_MESH = plsc.VectorSubcoreMesh(
    core_axis_name="c", subcore_axis_name="s", num_cores=NC, num_subcores=NS
)


@functools.partial(
    pl.kernel,
    out_type=jax.ShapeDtypeStruct((DHALF, NGRP, NLANE), jnp.float32),
    mesh=_SC_MESH,
    scratch_types=[
        pltpu.VMEM((CTX, NGRP, NLANE), jnp.int32),
        pltpu.VMEM((V,), jnp.float32),
        pltpu.VMEM((1, NGRP, NLANE), jnp.float32),
        pltpu.SemaphoreType.DMA,
        pltpu.SemaphoreType.DMA,
    ],
    compiler_params=pltpu.CompilerParams(
        needs_layout_passes=False, use_tc_tiling_on_sc=False
    ),
)
def _sc_pool(ctx_hbm, table_t_hbm, out_hbm, ctx_v, row_v, pooled_v, sem_c, sem_r):
    wid = lax.axis_index("s") * NC + lax.axis_index("c")
    c1 = pltpu.async_copy(ctx_hbm, ctx_v, sem_c)
    c2 = pltpu.async_copy(table_t_hbm.at[wid], row_v, sem_r)
    c1.wait()
    c2.wait()

    def group(g, carry):
        acc = plsc.load_gather(row_v, [ctx_v[0, g, :]])
        for c in range(1, CTX):
            acc = acc + plsc.load_gather(row_v, [ctx_v[c, g, :]])
        pooled_v[0, g, :] = acc * (1.0 / CTX)
        return carry

    lax.fori_loop(0, NGRP, group, 0)
    pltpu.sync_copy(pooled_v, out_hbm.at[pl.ds(wid, 1)])


VBLK = 2048
NVB = pl.cdiv(V, VBLK)


def _mm_body(wt_ref, pa_ref, pb_ref, b_ref, o_ref):
    dn = (((0,), (0,)), ((), ()))
    o_ref[...] = (
        lax.dot_general(wt_ref[0:DHALF], pa_ref[...], dn,
                        preferred_element_type=jnp.float32)
        + lax.dot_general(wt_ref[DHALF:D], pb_ref[...], dn,
                          preferred_element_type=jnp.float32)
        + b_ref[...].reshape(VBLK, 1)
    )


_matmul = pl.pallas_call(
    _mm_body,
    grid=(NVB,),
    in_specs=[
        pl.BlockSpec((D, VBLK), lambda v: (0, v)),
        pl.BlockSpec((DHALF, B), lambda v: (0, 0)),
        pl.BlockSpec((DHALF, B), lambda v: (0, 0)),
        pl.BlockSpec((VBLK,), lambda v: (v,)),
    ],
    out_specs=pl.BlockSpec((VBLK, B), lambda v: (v, 0)),
    out_shape=jax.ShapeDtypeStruct((V, B), jnp.float32),
    compiler_params=pltpu.CompilerParams(
        dimension_semantics=("parallel",),
        vmem_limit_bytes=100 * 1024 * 1024,
    ),
)


def kernel(context, emb_table, W, b):
    ctx_t = context.T.astype(jnp.int32).reshape(CTX, NGRP, NLANE)
    table_t = emb_table.T
    pooled_a = _sc_pool(ctx_t, table_t[:DHALF]).reshape(DHALF, B)
    pooled_b = _sc_pool(ctx_t, table_t[DHALF:]).reshape(DHALF, B)
    out_t = _matmul(W.T, pooled_a, pooled_b, b)
    return out_t.T


# speedup vs baseline: 1.1286x; 1.1286x over previous
import functools

import jax
import jax.numpy as jnp
from jax import lax
from jax.experimental import pallas as pl
from jax.experimental.pallas import tpu as pltpu
from jax.experimental.pallas import tpu_sc as plsc

V = 100000
D = 64
B = 1024
CTX = 20

NC = 2
NS = 16
NW = NC * NS
D_PER_W = D // NW
NLANE = 16
NGRP = B // NLANE

_SC_MESH = plsc.VectorSubcoreMesh(
    core_axis_name="c", subcore_axis_name="s", num_cores=NC, num_subcores=NS
)


@functools.partial(
    pl.kernel,
    out_type=jax.ShapeDtypeStruct((D, NGRP, NLANE), jnp.float32),
    mesh=_SC_MESH,
    scratch_types=[
        pltpu.VMEM((CTX, NGRP, NLANE), jnp.int32),
        pltpu.VMEM((V,), jnp.float32),
        pltpu.VMEM((D_PER_W, NGRP, NLANE), jnp.float32),
        pltpu.SemaphoreType.DMA,
        pltpu.SemaphoreType.DMA,
    ],
    compiler_params=pltpu.CompilerParams(
        needs_layout_passes=False, use_tc_tiling_on_sc=False
    ),
)
def _sc_pool(ctx_hbm, table_t_hbm, out_hbm, ctx_v, row_v, pooled_v, sem_c, sem_r):
    wid = lax.axis_index("s") * NC + lax.axis_index("c")
    cc = pltpu.async_copy(ctx_hbm, ctx_v, sem_c)
    rc = pltpu.async_copy(table_t_hbm.at[wid * D_PER_W], row_v, sem_r)
    cc.wait()
    for di in range(D_PER_W):
        d = wid * D_PER_W + di
        if di == 0:
            rc.wait()
        else:
            pltpu.sync_copy(table_t_hbm.at[d], row_v)

        def group(g, carry):
            acc = plsc.load_gather(row_v, [ctx_v[0, g, :]])
            for c in range(1, CTX):
                acc = acc + plsc.load_gather(row_v, [ctx_v[c, g, :]])
            pooled_v[di, g, :] = acc * (1.0 / CTX)
            return carry

        lax.fori_loop(0, NGRP, group, 0)
    pltpu.sync_copy(pooled_v, out_hbm.at[pl.ds(wid * D_PER_W, D_PER_W)])


VBLK = 2048
NVB = pl.cdiv(V, VBLK)


def _mm_body(wt_ref, p_ref, b_ref, o_ref):
    o_ref[...] = (
        lax.dot_general(
            wt_ref[...],
            p_ref[...],
            (((0,), (0,)), ((), ())),
            preferred_element_type=jnp.float32,
        )
        + b_ref[...].reshape(VBLK, 1)
    )


_matmul = pl.pallas_call(
    _mm_body,
    grid=(NVB,),
    in_specs=[
        pl.BlockSpec((D, VBLK), lambda v: (0, v)),
        pl.BlockSpec((D, B), lambda v: (0, 0)),
        pl.BlockSpec((VBLK,), lambda v: (v,)),
    ],
    out_specs=pl.BlockSpec((VBLK, B), lambda v: (v, 0)),
    out_shape=jax.ShapeDtypeStruct((V, B), jnp.float32),
    compiler_params=pltpu.CompilerParams(
        dimension_semantics=("parallel",),
        vmem_limit_bytes=100 * 1024 * 1024,
    ),
)


def kernel(context, emb_table, W, b):
    ctx_t = context.T.astype(jnp.int32).reshape(CTX, NGRP, NLANE)
    pooled_t = _sc_pool(ctx_t, emb_table.T).reshape(D, B)
    out_t = _matmul(W.T, pooled_t, b)
    return out_t.T


# speedup vs baseline: 1.1731x; 1.0394x over previous
import functools

import jax
import jax.numpy as jnp
from jax import lax
from jax.experimental import pallas as pl
from jax.experimental.pallas import tpu as pltpu
from jax.experimental.pallas import tpu_sc as plsc

V = 100000
D = 64
B = 1024
CTX = 20

NC = 2
NS = 16
NW = NC * NS
D_PER_W = D // NW
NLANE = 16
NGRP = B // NLANE

_SC_MESH = plsc.VectorSubcoreMesh(
    core_axis_name="c", subcore_axis_name="s", num_cores=NC, num_subcores=NS
)


@functools.partial(
    pl.kernel,
    out_type=jax.ShapeDtypeStruct((D, NGRP, NLANE), jnp.float32),
    mesh=_SC_MESH,
    scratch_types=[
        pltpu.VMEM((CTX, NGRP, NLANE), jnp.int32),
        pltpu.VMEM((V,), jnp.float32),
        pltpu.VMEM((D_PER_W, NGRP, NLANE), jnp.float32),
        pltpu.SemaphoreType.DMA,
        pltpu.SemaphoreType.DMA,
    ],
    compiler_params=pltpu.CompilerParams(
        needs_layout_passes=False, use_tc_tiling_on_sc=False
    ),
)
def _sc_pool(ctx_hbm, table_t_hbm, out_hbm, ctx_v, row_v, pooled_v, sem_c, sem_r):
    wid = lax.axis_index("s") * NC + lax.axis_index("c")
    cc = pltpu.async_copy(ctx_hbm, ctx_v, sem_c)
    rc = pltpu.async_copy(table_t_hbm.at[wid * D_PER_W], row_v, sem_r)
    cc.wait()
    for di in range(D_PER_W):
        d = wid * D_PER_W + di
        if di == 0:
            rc.wait()
        else:
            pltpu.sync_copy(table_t_hbm.at[d], row_v)

        def group(g, carry):
            acc = plsc.load_gather(row_v, [ctx_v[0, g, :]])
            for c in range(1, CTX):
                acc = acc + plsc.load_gather(row_v, [ctx_v[c, g, :]])
            pooled_v[di, g, :] = acc * (1.0 / CTX)
            return carry

        lax.fori_loop(0, NGRP, group, 0)
    pltpu.sync_copy(pooled_v, out_hbm.at[pl.ds(wid * D_PER_W, D_PER_W)])


VBLK = 2048
NVB = pl.cdiv(V, VBLK)


def _mm_body(wt_ref, p_ref, b_ref, o_ref):
    o_ref[...] = (
        lax.dot_general(
            wt_ref[...],
            p_ref[...].reshape(D, B),
            (((0,), (0,)), ((), ())),
            preferred_element_type=jnp.float32,
        )
        + b_ref[...].reshape(VBLK, 1)
    )


_matmul = pl.pallas_call(
    _mm_body,
    grid=(NVB,),
    in_specs=[
        pl.BlockSpec((D, VBLK), lambda v: (0, v)),
        pl.BlockSpec((D * B,), lambda v: (0,)),
        pl.BlockSpec((VBLK,), lambda v: (v,)),
    ],
    out_specs=pl.BlockSpec((VBLK, B), lambda v: (v, 0)),
    out_shape=jax.ShapeDtypeStruct((V, B), jnp.float32),
    compiler_params=pltpu.CompilerParams(
        dimension_semantics=("parallel",),
        vmem_limit_bytes=100 * 1024 * 1024,
    ),
)


def kernel(context, emb_table, W, b):
    ctx_t = context.T.astype(jnp.int32).reshape(CTX, NGRP, NLANE)
    pooled_t = _sc_pool(ctx_t, emb_table.T).reshape(D * B)
    out_t = _matmul(W.T, pooled_t, b)
    return out_t.T
